# Initial kernel scaffold; baseline (speedup 1.0000x reference)
import jax
import jax.numpy as jnp
from jax.experimental import pallas as pl


def kernel(x, edge_index, node_graph_ids, gin_W, gin_b, gat_W, gat_al, gat_ar, gat_b, lstm_Wih, lstm_Whh, lstm_bih, lstm_bhh, tw1, tb1, tw2, tb2, tw3, tb3):
    raise NotImplementedError("write your pallas kernel here")


# speedup vs baseline: 20.2233x; 20.2233x over previous
import functools

import jax
import jax.numpy as jnp
from jax import lax
from jax.experimental import pallas as pl
from jax.experimental.pallas import tpu as pltpu
from jax.experimental.pallas import tpu_sc as plsc

N_NODES = 10000
N_EDGES = 320000
NUM_GRAPHS = 64
D = 42
H = 6
HP = 48
S2S = 84

NUM_SUBCORES = 16
EDGES_PER_SUBCORE = N_EDGES // NUM_SUBCORES
CHUNK = 80
NCHUNKS = EDGES_PER_SUBCORE // CHUNK
NP = 10240


def _prep_body(x_ref, w_ref, al_ref, ar_ref,
               f0_ref, f1_ref, el_ref, er_ref, xp_ref):
    x = x_ref[...]
    feat = jnp.dot(x, w_ref[...], preferred_element_type=jnp.float32)
    r = x.shape[0]
    ones = jnp.ones((r, 1), jnp.float32)
    zer5 = jnp.zeros((r, 5), jnp.float32)
    al = al_ref[...]
    ar = ar_ref[...]
    blocks, els, ers = [], [], []
    for h in range(H):
        fh = feat[:, D * h:D * (h + 1)]
        els.append(jnp.sum(fh * al[h][None, :], axis=1, keepdims=True))
        ers.append(jnp.sum(fh * ar[h][None, :], axis=1, keepdims=True))
        blocks.append(jnp.concatenate([fh, ones, zer5], axis=1))
    f0_ref[...] = jnp.concatenate(blocks[:3], axis=1)
    f1_ref[...] = jnp.concatenate(blocks[3:], axis=1)
    pad10 = jnp.zeros((r, 10), jnp.float32)
    el_ref[...] = jnp.concatenate(els + [pad10], axis=1)
    er_ref[...] = jnp.concatenate(ers + [pad10], axis=1)
    xp_ref[...] = jnp.concatenate([x, ones, zer5], axis=1)


def _prep(x, gat_W, gat_al, gat_ar):
    rb = 1000
    grid = N_NODES // rb
    return pl.pallas_call(
        _prep_body,
        grid=(grid,),
        in_specs=[
            pl.BlockSpec((rb, D), lambda i: (i, 0)),
            pl.BlockSpec((D, H * D), lambda i: (0, 0)),
            pl.BlockSpec((H, D), lambda i: (0, 0)),
            pl.BlockSpec((H, D), lambda i: (0, 0)),
        ],
        out_specs=[
            pl.BlockSpec((rb, 3 * HP), lambda i: (i, 0)),
            pl.BlockSpec((rb, 3 * HP), lambda i: (i, 0)),
            pl.BlockSpec((rb, 16), lambda i: (i, 0)),
            pl.BlockSpec((rb, 16), lambda i: (i, 0)),
            pl.BlockSpec((rb, HP), lambda i: (i, 0)),
        ],
        out_shape=[
            jax.ShapeDtypeStruct((N_NODES, 3 * HP), jnp.float32),
            jax.ShapeDtypeStruct((N_NODES, 3 * HP), jnp.float32),
            jax.ShapeDtypeStruct((N_NODES, 16), jnp.float32),
            jax.ShapeDtypeStruct((N_NODES, 16), jnp.float32),
            jax.ShapeDtypeStruct((N_NODES, HP), jnp.float32),
        ],
    )(x, gat_W, gat_al, gat_ar)


def _sc_gat_body(src_hbm, dst_hbm, elp_hbm, erp_hbm, fcat_hbm, z144_hbm,
                 outcat_hbm,
                 srcv, dstv, srcadj, els, ers, feats, vals,
                 out_sh, sem1, sem2, sem3):
    cid = lax.axis_index("c")
    sid = lax.axis_index("s")

    rows = NP // NUM_SUBCORES
    r0 = sid * rows
    pltpu.sync_copy(z144_hbm.at[pl.ds(r0, rows)], out_sh.at[pl.ds(r0, rows)])
    plsc.subcore_barrier()

    lanes = lax.iota(jnp.int32, 16)
    h0 = 3 * cid
    zero16 = jnp.zeros((16,), jnp.int32)
    hidx = [(zero16 + (h0 + hh))[:, None] for hh in range(3)]
    dnums = lax.GatherDimensionNumbers(
        offset_dims=(), collapsed_slice_dims=(0,), start_index_map=(0,))
    row_off = cid * N_NODES
    out_off = cid * NP

    base_e = sid * EDGES_PER_SUBCORE

    def chunk_body(k, carry):
        cb = base_e + k * CHUNK
        pltpu.sync_copy(src_hbm.at[pl.ds(cb, CHUNK)], srcv)
        pltpu.sync_copy(dst_hbm.at[pl.ds(cb, CHUNK)], dstv)
        for i in range(CHUNK // 16):
            srcadj[pl.ds(i * 16, 16)] = srcv[pl.ds(i * 16, 16)] + row_off
        cp1 = pltpu.async_copy(fcat_hbm.at[srcadj], feats, sem1)
        cp2 = pltpu.async_copy(elp_hbm.at[srcv], els, sem2)
        cp3 = pltpu.async_copy(erp_hbm.at[dstv], ers, sem3)
        cp2.wait()
        cp3.wait()
        cp1.wait()

        def edge_body(i, c2):
            e = els[i] + ers[i]
            e = jnp.where(e >= 0.0, e, 0.2 * e)
            ex = jnp.exp(e)
            exb = [lax.gather(ex, hidx[hh], dnums, slice_sizes=(1,),
                              mode=lax.GatherScatterMode.PROMISE_IN_BOUNDS)
                   for hh in range(3)]
            for j in range(9):
                vals[i, pl.ds(16 * j, 16)] = (
                    exb[j // 3] * feats[i, pl.ds(16 * j, 16)])
            return c2

        lax.fori_loop(0, CHUNK, edge_body, 0)

        pltpu.sync_copy(vals, out_sh.at[dstv], add=True)
        return carry

    lax.fori_loop(0, NCHUNKS, chunk_body, 0)
    plsc.subcore_barrier()

    pltpu.sync_copy(out_sh.at[pl.ds(r0, rows)],
                    outcat_hbm.at[pl.ds(out_off + r0, rows)])


def _sc_gin_body(src_hbm, dst_hbm, xp_hbm, z48_hbm,
                 agg_hbm,
                 srcv, dstv, xs,
                 agg_sh, sem1):
    cid = lax.axis_index("c")
    sid = lax.axis_index("s")

    rows = NP // NUM_SUBCORES
    r0 = sid * rows
    pltpu.sync_copy(z48_hbm.at[pl.ds(r0, rows)], agg_sh.at[pl.ds(r0, rows)])
    plsc.subcore_barrier()

    half = N_EDGES // 2
    ept = half // NUM_SUBCORES
    base_e = cid * half + sid * ept
    out_off = cid * NP

    def chunk_body(k, carry):
        cb = base_e + k * CHUNK
        pltpu.sync_copy(src_hbm.at[pl.ds(cb, CHUNK)], srcv)
        pltpu.sync_copy(dst_hbm.at[pl.ds(cb, CHUNK)], dstv)
        pltpu.async_copy(xp_hbm.at[srcv], xs, sem1).wait()
        pltpu.sync_copy(xs, agg_sh.at[dstv], add=True)
        return carry

    lax.fori_loop(0, ept // CHUNK, chunk_body, 0)
    plsc.subcore_barrier()

    pltpu.sync_copy(agg_sh.at[pl.ds(r0, rows)],
                    agg_hbm.at[pl.ds(out_off + r0, rows)])


def _sc_edge(src, dst, elp, erp, fcat, xp, z144, z48):
    mesh = plsc.VectorSubcoreMesh(core_axis_name="c", subcore_axis_name="s")
    gat = functools.partial(
        pl.kernel,
        mesh=mesh,
        compiler_params=pltpu.CompilerParams(use_tc_tiling_on_sc=False),
        out_type=jax.ShapeDtypeStruct((2 * NP, 3 * HP), jnp.float32),
        scratch_types=[
            pltpu.VMEM((CHUNK,), jnp.int32),
            pltpu.VMEM((CHUNK,), jnp.int32),
            pltpu.VMEM((CHUNK,), jnp.int32),
            pltpu.VMEM((CHUNK, 16), jnp.float32),
            pltpu.VMEM((CHUNK, 16), jnp.float32),
            pltpu.VMEM((CHUNK, 3 * HP), jnp.float32),
            pltpu.VMEM((CHUNK, 3 * HP), jnp.float32),
            pltpu.VMEM_SHARED((NP, 3 * HP), jnp.float32),
            pltpu.SemaphoreType.DMA,
            pltpu.SemaphoreType.DMA,
            pltpu.SemaphoreType.DMA,
        ],
    )(_sc_gat_body)
    gin = functools.partial(
        pl.kernel,
        mesh=mesh,
        compiler_params=pltpu.CompilerParams(use_tc_tiling_on_sc=False),
        out_type=jax.ShapeDtypeStruct((2 * NP, HP), jnp.float32),
        scratch_types=[
            pltpu.VMEM((CHUNK,), jnp.int32),
            pltpu.VMEM((CHUNK,), jnp.int32),
            pltpu.VMEM((CHUNK, HP), jnp.float32),
            pltpu.VMEM_SHARED((NP, HP), jnp.float32),
            pltpu.SemaphoreType.DMA,
        ],
    )(_sc_gin_body)
    outcat = gat(src, dst, elp, erp, fcat, z144)
    aggcat = gin(src, dst, xp, z48)
    return outcat, aggcat


def _nodes_body(x_ref, agg_ref, out0_ref, out1_ref, ginw_ref, ginb_ref,
                gatb_ref, z_ref):
    x = x_ref[...]
    agg = agg_ref[...]
    deg = agg[:, D:D + 1]
    mean = jnp.where(deg > 0.0, agg[:, :D] / jnp.maximum(deg, 1.0), 0.0)
    gin = jnp.dot(x + mean, ginw_ref[...],
                  preferred_element_type=jnp.float32) + ginb_ref[...][None, :]
    gatb = gatb_ref[...]
    for t in range(H):
        oref = out0_ref if t < 3 else out1_ref
        c0 = HP * (t % 3)
        blk = oref[:, c0:c0 + HP]
        denom = jnp.maximum(blk[:, D:D + 1], 1e-9)
        gat = blk[:, :D] / denom + gatb[t][None, :]
        z_ref[t] = jnp.concatenate([gin, gat], axis=1)


def _nodes(x, agg, out0, out1, gin_W, gin_b, gat_b):
    rb = 1000
    return pl.pallas_call(
        _nodes_body,
        grid=(N_NODES // rb,),
        in_specs=[
            pl.BlockSpec((rb, D), lambda i: (i, 0)),
            pl.BlockSpec((rb, HP), lambda i: (i, 0)),
            pl.BlockSpec((rb, 3 * HP), lambda i: (i, 0)),
            pl.BlockSpec((rb, 3 * HP), lambda i: (i, 0)),
            pl.BlockSpec((D, D), lambda i: (0, 0)),
            pl.BlockSpec((D,), lambda i: (0,)),
            pl.BlockSpec((H, D), lambda i: (0, 0)),
        ],
        out_specs=pl.BlockSpec((H, rb, 2 * D), lambda i: (0, i, 0)),
        out_shape=jax.ShapeDtypeStruct((H, N_NODES, 2 * D), jnp.float32),
    )(x, agg, out0, out1, gin_W, gin_b, gat_b)


def _pool_body(z_ref, gid_col_ref, gid_row_ref,
               wih_ref, whh_ref, bih_ref, bhh_ref,
               tw1_ref, tb1_ref, tw2_ref, tb2_ref, tw3_ref, tb3_ref,
               y_ref):
    z = z_ref[0]
    gid_col = gid_col_ref[...]
    gid_row = gid_row_ref[...]
    oh_b = lax.broadcasted_iota(jnp.int32, (N_NODES, NUM_GRAPHS), 1) == gid_col
    oh = oh_b.astype(jnp.float32)
    oht = (lax.broadcasted_iota(jnp.int32, (NUM_GRAPHS, N_NODES), 0)
           == gid_row).astype(jnp.float32)

    wih = wih_ref[...]
    whh = whh_ref[...]
    bih = bih_ref[...][None, :]
    bhh = bhh_ref[...][None, :]
    neg_inf = jnp.float32(-jnp.inf)

    h = jnp.zeros((NUM_GRAPHS, S2S), jnp.float32)
    c = jnp.zeros((NUM_GRAPHS, S2S), jnp.float32)
    q_star = jnp.zeros((NUM_GRAPHS, 2 * S2S), jnp.float32)
    for _ in range(2):
        gates = (lax.dot_general(q_star, wih, (((1,), (1,)), ((), ())),
                                 preferred_element_type=jnp.float32)
                 + bih
                 + lax.dot_general(h, whh, (((1,), (1,)), ((), ())),
                                   preferred_element_type=jnp.float32)
                 + bhh)
        ig = jax.nn.sigmoid(gates[:, :S2S])
        fg = jax.nn.sigmoid(gates[:, S2S:2 * S2S])
        gg = jnp.tanh(gates[:, 2 * S2S:3 * S2S])
        og = jax.nn.sigmoid(gates[:, 3 * S2S:])
        c = fg * c + ig * gg
        h = og * jnp.tanh(c)
        q = h
        e1 = lax.dot_general(z, q, (((1,), (1,)), ((), ())),
                             preferred_element_type=jnp.float32)
        e = jnp.sum(e1 * oh, axis=1, keepdims=True)
        m = jnp.max(jnp.where(oh_b, e1, neg_inf), axis=0,
                    keepdims=True)
        m = jnp.where(jnp.isfinite(m), m, 0.0)
        mgid = jnp.sum(oh * m, axis=1, keepdims=True)
        ex = jnp.exp(e - mgid)
        s = jnp.sum(oh * ex, axis=0, keepdims=True)
        sgid = jnp.sum(oh * s, axis=1, keepdims=True)
        alpha = ex / jnp.maximum(sgid, 1e-9)
        r = jnp.dot(oht, alpha * z,
                    preferred_element_type=jnp.float32)
        q_star = jnp.concatenate([q, r], axis=1)

    h1 = q_star @ tw1_ref[0] + tb1_ref[0]
    h1 = jnp.where(h1 > 0.0, h1, jnp.exp(jnp.minimum(h1, 0.0)) - 1.0)
    h2 = h1 @ tw2_ref[0] + tb2_ref[0]
    h2 = jnp.where(h2 > 0.0, h2, jnp.exp(jnp.minimum(h2, 0.0)) - 1.0)
    y_ref[0] = h2 @ tw3_ref[0] + tb3_ref[0]


def _pool(zstack, gid_col, gid_row, lstm_Wih, lstm_Whh, lstm_bih, lstm_bhh,
          tw1, tb1, tw2, tb2, tw3, tb3):
    full = lambda *s: pl.BlockSpec(s, lambda t: tuple(0 for _ in s))
    per_task = lambda *s: pl.BlockSpec((1,) + s, lambda t: (t,) + tuple(
        0 for _ in s))
    return pl.pallas_call(
        _pool_body,
        grid=(H,),
        in_specs=[
            per_task(N_NODES, 2 * D),
            full(N_NODES, 1),
            full(1, N_NODES),
            full(4 * S2S, 2 * S2S),
            full(4 * S2S, S2S),
            full(4 * S2S),
            full(4 * S2S),
            per_task(2 * S2S, 32),
            per_task(1, 32),
            per_task(32, 16),
            per_task(1, 16),
            per_task(16, 1),
            per_task(1, 1),
        ],
        out_specs=per_task(NUM_GRAPHS, 1),
        out_shape=jax.ShapeDtypeStruct((H, NUM_GRAPHS, 1), jnp.float32),
    )(zstack, gid_col, gid_row, lstm_Wih, lstm_Whh, lstm_bih, lstm_bhh,
      tw1, tb1, tw2, tb2, tw3, tb3)


def kernel(x, edge_index, node_graph_ids, gin_W, gin_b, gat_W, gat_al, gat_ar,
           gat_b, lstm_Wih, lstm_Whh, lstm_bih, lstm_bhh,
           tw1, tb1, tw2, tb2, tw3, tb3):
    src = edge_index[0]
    dst = edge_index[1]
    f0, f1, elp, erp, xp = _prep(x, gat_W, gat_al, gat_ar)
    fcat = jnp.concatenate([f0, f1], axis=0)
    z144 = jnp.zeros((NP, 3 * HP), jnp.float32)
    z48 = jnp.zeros((NP, HP), jnp.float32)
    outcat, aggcat = _sc_edge(src, dst, elp, erp, fcat, xp, z144, z48)
    out0 = outcat[:N_NODES]
    out1 = outcat[NP:NP + N_NODES]
    agg = aggcat[:N_NODES] + aggcat[NP:NP + N_NODES]
    gid_col = node_graph_ids.reshape(N_NODES, 1)
    gid_row = node_graph_ids.reshape(1, N_NODES)
    zstack = _nodes(x, agg, out0, out1, gin_W, gin_b, gat_b)
    y = _pool(zstack, gid_col, gid_row, lstm_Wih, lstm_Whh, lstm_bih,
              lstm_bhh, tw1, tb1.reshape(H, 1, 32), tw2, tb2.reshape(H, 1, 16),
              tw3, tb3.reshape(H, 1, 1))
    return y.reshape(H * NUM_GRAPHS, 1)


# speedup vs baseline: 35.6545x; 1.7630x over previous
import functools

import jax
import jax.numpy as jnp
from jax import lax
from jax.experimental import pallas as pl
from jax.experimental.pallas import tpu as pltpu
from jax.experimental.pallas import tpu_sc as plsc

N_NODES = 10000
N_EDGES = 320000
NUM_GRAPHS = 64
D = 42
H = 6
HP = 48
S2S = 84

NUM_SUBCORES = 16
EDGES_PER_SUBCORE = N_EDGES // NUM_SUBCORES
CHUNK = 80
NCHUNKS = EDGES_PER_SUBCORE // CHUNK
NP = 10240


def _prep_body(x_ref, w_ref, al_ref, ar_ref,
               f0_ref, f1_ref, el_ref, er_ref, xp_ref):
    x = x_ref[...]
    feat = jnp.dot(x, w_ref[...], preferred_element_type=jnp.float32)
    r = x.shape[0]
    ones = jnp.ones((r, 1), jnp.float32)
    zer5 = jnp.zeros((r, 5), jnp.float32)
    al = al_ref[...]
    ar = ar_ref[...]
    blocks, els, ers = [], [], []
    for h in range(H):
        fh = feat[:, D * h:D * (h + 1)]
        els.append(jnp.sum(fh * al[h][None, :], axis=1, keepdims=True))
        ers.append(jnp.sum(fh * ar[h][None, :], axis=1, keepdims=True))
        blocks.append(jnp.concatenate([fh, ones, zer5], axis=1))
    f0_ref[...] = jnp.concatenate(blocks[:3], axis=1)
    f1_ref[...] = jnp.concatenate(blocks[3:], axis=1)
    pad10 = jnp.zeros((r, 10), jnp.float32)
    el_ref[...] = jnp.concatenate(els + [pad10], axis=1)
    er_ref[...] = jnp.concatenate(ers + [pad10], axis=1)
    xp_ref[...] = jnp.concatenate([x, ones, zer5], axis=1)


def _prep(x, gat_W, gat_al, gat_ar):
    rb = 1000
    grid = N_NODES // rb
    return pl.pallas_call(
        _prep_body,
        grid=(grid,),
        in_specs=[
            pl.BlockSpec((rb, D), lambda i: (i, 0)),
            pl.BlockSpec((D, H * D), lambda i: (0, 0)),
            pl.BlockSpec((H, D), lambda i: (0, 0)),
            pl.BlockSpec((H, D), lambda i: (0, 0)),
        ],
        out_specs=[
            pl.BlockSpec((rb, 3 * HP), lambda i: (i, 0)),
            pl.BlockSpec((rb, 3 * HP), lambda i: (i, 0)),
            pl.BlockSpec((rb, 16), lambda i: (i, 0)),
            pl.BlockSpec((rb, 16), lambda i: (i, 0)),
            pl.BlockSpec((rb, HP), lambda i: (i, 0)),
        ],
        out_shape=[
            jax.ShapeDtypeStruct((N_NODES, 3 * HP), jnp.float32),
            jax.ShapeDtypeStruct((N_NODES, 3 * HP), jnp.float32),
            jax.ShapeDtypeStruct((N_NODES, 16), jnp.float32),
            jax.ShapeDtypeStruct((N_NODES, 16), jnp.float32),
            jax.ShapeDtypeStruct((N_NODES, HP), jnp.float32),
        ],
    )(x, gat_W, gat_al, gat_ar)


def _sc_gat_body(src_hbm, dst_hbm, elp_hbm, erp_hbm, fcat_hbm, z144_hbm,
                 outcat_hbm,
                 srcva, dstva, adja, srcvb, dstvb, adjb,
                 elsa, ersa, elsb, ersb, featsa, featsb,
                 out_sh, g1a, g2a, g3a, g1b, g2b, g3b, ssa, ssb):
    cid = lax.axis_index("c")
    sid = lax.axis_index("s")

    rows = NP // NUM_SUBCORES
    r0 = sid * rows
    pltpu.sync_copy(z144_hbm.at[pl.ds(r0, rows)], out_sh.at[pl.ds(r0, rows)])
    plsc.subcore_barrier()

    h0 = 3 * cid
    zero16 = jnp.zeros((16,), jnp.int32)
    hidx = [(zero16 + (h0 + hh))[:, None] for hh in range(3)]
    dnums = lax.GatherDimensionNumbers(
        offset_dims=(), collapsed_slice_dims=(0,), start_index_map=(0,))
    row_off = cid * N_NODES
    out_off = cid * NP

    base_e = sid * EDGES_PER_SUBCORE

    def fire_gathers(cb, srcv, dstv, adj, els, ers, feats, s1, s2, s3):
        pltpu.sync_copy(src_hbm.at[pl.ds(cb, CHUNK)], srcv)
        pltpu.sync_copy(dst_hbm.at[pl.ds(cb, CHUNK)], dstv)
        for i in range(CHUNK // 16):
            adj[pl.ds(i * 16, 16)] = srcv[pl.ds(i * 16, 16)] + row_off
        cp1 = pltpu.async_copy(fcat_hbm.at[adj], feats, s1)
        cp2 = pltpu.async_copy(elp_hbm.at[srcv], els, s2)
        cp3 = pltpu.async_copy(erp_hbm.at[dstv], ers, s3)
        return cp1, cp2, cp3

    def compute(els, ers, feats):
        def edge_body(i4, c2):
            for u in range(4):
                i = 4 * i4 + u
                e = els[i] + ers[i]
                e = jnp.where(e >= 0.0, e, 0.2 * e)
                ex = jnp.exp(e)
                exb = [lax.gather(ex, hidx[hh], dnums, slice_sizes=(1,),
                                  mode=lax.GatherScatterMode.PROMISE_IN_BOUNDS)
                       for hh in range(3)]
                for j in range(9):
                    feats[i, pl.ds(16 * j, 16)] = (
                        exb[j // 3] * feats[i, pl.ds(16 * j, 16)])
            return c2

        lax.fori_loop(0, CHUNK // 4, edge_body, 0)

    def pair_body(k, carry):
        cb = base_e + 2 * k * CHUNK
        cpa = fire_gathers(cb, srcva, dstva, adja, elsa, ersa, featsa,
                           g1a, g2a, g3a)
        cpb = fire_gathers(cb + CHUNK, srcvb, dstvb, adjb, elsb, ersb, featsb,
                           g1b, g2b, g3b)
        for cp in cpa:
            cp.wait()
        compute(elsa, ersa, featsa)
        sa = pltpu.async_copy(featsa, out_sh.at[dstva], ssa, add=True)
        for cp in cpb:
            cp.wait()
        compute(elsb, ersb, featsb)
        sb = pltpu.async_copy(featsb, out_sh.at[dstvb], ssb, add=True)
        sa.wait()
        sb.wait()
        return carry

    lax.fori_loop(0, NCHUNKS // 2, pair_body, 0)
    plsc.subcore_barrier()

    pltpu.sync_copy(out_sh.at[pl.ds(r0, rows)],
                    outcat_hbm.at[pl.ds(out_off + r0, rows)])


def _sc_gin_body(src_hbm, dst_hbm, xp_hbm, z48_hbm,
                 agg_hbm,
                 srcv, dstv, xs,
                 agg_sh, sem1):
    cid = lax.axis_index("c")
    sid = lax.axis_index("s")

    rows = NP // NUM_SUBCORES
    r0 = sid * rows
    pltpu.sync_copy(z48_hbm.at[pl.ds(r0, rows)], agg_sh.at[pl.ds(r0, rows)])
    plsc.subcore_barrier()

    half = N_EDGES // 2
    ept = half // NUM_SUBCORES
    base_e = cid * half + sid * ept
    out_off = cid * NP

    def chunk_body(k, carry):
        cb = base_e + k * CHUNK
        pltpu.sync_copy(src_hbm.at[pl.ds(cb, CHUNK)], srcv)
        pltpu.sync_copy(dst_hbm.at[pl.ds(cb, CHUNK)], dstv)
        pltpu.async_copy(xp_hbm.at[srcv], xs, sem1).wait()
        pltpu.sync_copy(xs, agg_sh.at[dstv], add=True)
        return carry

    lax.fori_loop(0, ept // CHUNK, chunk_body, 0)
    plsc.subcore_barrier()

    pltpu.sync_copy(agg_sh.at[pl.ds(r0, rows)],
                    agg_hbm.at[pl.ds(out_off + r0, rows)])


def _sc_edge(src, dst, elp, erp, fcat, xp, z144, z48):
    mesh = plsc.VectorSubcoreMesh(core_axis_name="c", subcore_axis_name="s")
    gat = functools.partial(
        pl.kernel,
        mesh=mesh,
        compiler_params=pltpu.CompilerParams(use_tc_tiling_on_sc=False),
        out_type=jax.ShapeDtypeStruct((2 * NP, 3 * HP), jnp.float32),
        scratch_types=(
            [pltpu.VMEM((CHUNK,), jnp.int32)] * 6
            + [pltpu.VMEM((CHUNK, 16), jnp.float32)] * 4
            + [pltpu.VMEM((CHUNK, 3 * HP), jnp.float32)] * 2
            + [pltpu.VMEM_SHARED((NP, 3 * HP), jnp.float32)]
            + [pltpu.SemaphoreType.DMA] * 8
        ),
    )(_sc_gat_body)
    gin = functools.partial(
        pl.kernel,
        mesh=mesh,
        compiler_params=pltpu.CompilerParams(use_tc_tiling_on_sc=False),
        out_type=jax.ShapeDtypeStruct((2 * NP, HP), jnp.float32),
        scratch_types=[
            pltpu.VMEM((CHUNK,), jnp.int32),
            pltpu.VMEM((CHUNK,), jnp.int32),
            pltpu.VMEM((CHUNK, HP), jnp.float32),
            pltpu.VMEM_SHARED((NP, HP), jnp.float32),
            pltpu.SemaphoreType.DMA,
        ],
    )(_sc_gin_body)
    outcat = gat(src, dst, elp, erp, fcat, z144)
    aggcat = gin(src, dst, xp, z48)
    return outcat, aggcat


def _nodes_body(x_ref, agg_ref, out0_ref, out1_ref, ginw_ref, ginb_ref,
                gatb_ref, z_ref):
    x = x_ref[...]
    agg = agg_ref[...]
    deg = agg[:, D:D + 1]
    mean = jnp.where(deg > 0.0, agg[:, :D] / jnp.maximum(deg, 1.0), 0.0)
    gin = jnp.dot(x + mean, ginw_ref[...],
                  preferred_element_type=jnp.float32) + ginb_ref[...][None, :]
    gatb = gatb_ref[...]
    for t in range(H):
        oref = out0_ref if t < 3 else out1_ref
        c0 = HP * (t % 3)
        blk = oref[:, c0:c0 + HP]
        denom = jnp.maximum(blk[:, D:D + 1], 1e-9)
        gat = blk[:, :D] / denom + gatb[t][None, :]
        z_ref[t] = jnp.concatenate([gin, gat], axis=1)


def _nodes(x, agg, out0, out1, gin_W, gin_b, gat_b):
    rb = 1000
    return pl.pallas_call(
        _nodes_body,
        grid=(N_NODES // rb,),
        in_specs=[
            pl.BlockSpec((rb, D), lambda i: (i, 0)),
            pl.BlockSpec((rb, HP), lambda i: (i, 0)),
            pl.BlockSpec((rb, 3 * HP), lambda i: (i, 0)),
            pl.BlockSpec((rb, 3 * HP), lambda i: (i, 0)),
            pl.BlockSpec((D, D), lambda i: (0, 0)),
            pl.BlockSpec((D,), lambda i: (0,)),
            pl.BlockSpec((H, D), lambda i: (0, 0)),
        ],
        out_specs=pl.BlockSpec((H, rb, 2 * D), lambda i: (0, i, 0)),
        out_shape=jax.ShapeDtypeStruct((H, N_NODES, 2 * D), jnp.float32),
    )(x, agg, out0, out1, gin_W, gin_b, gat_b)


def _pool_body(z_ref, gid_col_ref, gid_row_ref,
               wih_ref, whh_ref, bih_ref, bhh_ref,
               tw1_ref, tb1_ref, tw2_ref, tb2_ref, tw3_ref, tb3_ref,
               y_ref):
    z = z_ref[0]
    gid_col = gid_col_ref[...]
    gid_row = gid_row_ref[...]
    oh_b = lax.broadcasted_iota(jnp.int32, (N_NODES, NUM_GRAPHS), 1) == gid_col
    oh = oh_b.astype(jnp.float32)
    oht = (lax.broadcasted_iota(jnp.int32, (NUM_GRAPHS, N_NODES), 0)
           == gid_row).astype(jnp.float32)

    wih = wih_ref[...]
    whh = whh_ref[...]
    bih = bih_ref[...][None, :]
    bhh = bhh_ref[...][None, :]
    neg_inf = jnp.float32(-jnp.inf)

    h = jnp.zeros((NUM_GRAPHS, S2S), jnp.float32)
    c = jnp.zeros((NUM_GRAPHS, S2S), jnp.float32)
    q_star = jnp.zeros((NUM_GRAPHS, 2 * S2S), jnp.float32)
    for _ in range(2):
        gates = (lax.dot_general(q_star, wih, (((1,), (1,)), ((), ())),
                                 preferred_element_type=jnp.float32)
                 + bih
                 + lax.dot_general(h, whh, (((1,), (1,)), ((), ())),
                                   preferred_element_type=jnp.float32)
                 + bhh)
        ig = jax.nn.sigmoid(gates[:, :S2S])
        fg = jax.nn.sigmoid(gates[:, S2S:2 * S2S])
        gg = jnp.tanh(gates[:, 2 * S2S:3 * S2S])
        og = jax.nn.sigmoid(gates[:, 3 * S2S:])
        c = fg * c + ig * gg
        h = og * jnp.tanh(c)
        q = h
        e1 = lax.dot_general(z, q, (((1,), (1,)), ((), ())),
                             preferred_element_type=jnp.float32)
        e = jnp.sum(e1 * oh, axis=1, keepdims=True)
        m = jnp.max(jnp.where(oh_b, e1, neg_inf), axis=0,
                    keepdims=True)
        m = jnp.where(jnp.isfinite(m), m, 0.0)
        mgid = jnp.sum(oh * m, axis=1, keepdims=True)
        ex = jnp.exp(e - mgid)
        s = jnp.sum(oh * ex, axis=0, keepdims=True)
        sgid = jnp.sum(oh * s, axis=1, keepdims=True)
        alpha = ex / jnp.maximum(sgid, 1e-9)
        r = jnp.dot(oht, alpha * z,
                    preferred_element_type=jnp.float32)
        q_star = jnp.concatenate([q, r], axis=1)

    h1 = q_star @ tw1_ref[0] + tb1_ref[0]
    h1 = jnp.where(h1 > 0.0, h1, jnp.exp(jnp.minimum(h1, 0.0)) - 1.0)
    h2 = h1 @ tw2_ref[0] + tb2_ref[0]
    h2 = jnp.where(h2 > 0.0, h2, jnp.exp(jnp.minimum(h2, 0.0)) - 1.0)
    y_ref[0] = h2 @ tw3_ref[0] + tb3_ref[0]


def _pool(zstack, gid_col, gid_row, lstm_Wih, lstm_Whh, lstm_bih, lstm_bhh,
          tw1, tb1, tw2, tb2, tw3, tb3):
    full = lambda *s: pl.BlockSpec(s, lambda t: tuple(0 for _ in s))
    per_task = lambda *s: pl.BlockSpec((1,) + s, lambda t: (t,) + tuple(
        0 for _ in s))
    return pl.pallas_call(
        _pool_body,
        grid=(H,),
        in_specs=[
            per_task(N_NODES, 2 * D),
            full(N_NODES, 1),
            full(1, N_NODES),
            full(4 * S2S, 2 * S2S),
            full(4 * S2S, S2S),
            full(4 * S2S),
            full(4 * S2S),
            per_task(2 * S2S, 32),
            per_task(1, 32),
            per_task(32, 16),
            per_task(1, 16),
            per_task(16, 1),
            per_task(1, 1),
        ],
        out_specs=per_task(NUM_GRAPHS, 1),
        out_shape=jax.ShapeDtypeStruct((H, NUM_GRAPHS, 1), jnp.float32),
    )(zstack, gid_col, gid_row, lstm_Wih, lstm_Whh, lstm_bih, lstm_bhh,
      tw1, tb1, tw2, tb2, tw3, tb3)


def kernel(x, edge_index, node_graph_ids, gin_W, gin_b, gat_W, gat_al, gat_ar,
           gat_b, lstm_Wih, lstm_Whh, lstm_bih, lstm_bhh,
           tw1, tb1, tw2, tb2, tw3, tb3):
    src = edge_index[0]
    dst = edge_index[1]
    f0, f1, elp, erp, xp = _prep(x, gat_W, gat_al, gat_ar)
    fcat = jnp.concatenate([f0, f1], axis=0)
    z144 = jnp.zeros((NP, 3 * HP), jnp.float32)
    z48 = jnp.zeros((NP, HP), jnp.float32)
    outcat, aggcat = _sc_edge(src, dst, elp, erp, fcat, xp, z144, z48)
    out0 = outcat[:N_NODES]
    out1 = outcat[NP:NP + N_NODES]
    agg = aggcat[:N_NODES] + aggcat[NP:NP + N_NODES]
    gid_col = node_graph_ids.reshape(N_NODES, 1)
    gid_row = node_graph_ids.reshape(1, N_NODES)
    zstack = _nodes(x, agg, out0, out1, gin_W, gin_b, gat_b)
    y = _pool(zstack, gid_col, gid_row, lstm_Wih, lstm_Whh, lstm_bih,
              lstm_bhh, tw1, tb1.reshape(H, 1, 32), tw2, tb2.reshape(H, 1, 16),
              tw3, tb3.reshape(H, 1, 1))
    return y.reshape(H * NUM_GRAPHS, 1)


# speedup vs baseline: 37.4650x; 1.0508x over previous
import functools

import jax
import jax.numpy as jnp
from jax import lax
from jax.experimental import pallas as pl
from jax.experimental.pallas import tpu as pltpu
from jax.experimental.pallas import tpu_sc as plsc

N_NODES = 10000
N_EDGES = 320000
NUM_GRAPHS = 64
D = 42
H = 6
HP = 48
S2S = 84

NUM_SUBCORES = 16
EDGES_PER_SUBCORE = N_EDGES // NUM_SUBCORES
CHUNK = 80
NCHUNKS = EDGES_PER_SUBCORE // CHUNK
NP = 10240


def _prep_body(x_ref, w_ref, al_ref, ar_ref,
               f0_ref, f1_ref, el_ref, er_ref, xp_ref):
    x = x_ref[...]
    feat = jnp.dot(x, w_ref[...], preferred_element_type=jnp.float32)
    r = x.shape[0]
    ones = jnp.ones((r, 1), jnp.float32)
    zer5 = jnp.zeros((r, 5), jnp.float32)
    al = al_ref[...]
    ar = ar_ref[...]
    blocks, els, ers = [], [], []
    for h in range(H):
        fh = feat[:, D * h:D * (h + 1)]
        els.append(jnp.sum(fh * al[h][None, :], axis=1, keepdims=True))
        ers.append(jnp.sum(fh * ar[h][None, :], axis=1, keepdims=True))
        blocks.append(jnp.concatenate([fh, ones, zer5], axis=1))
    f0_ref[...] = jnp.concatenate(blocks[:3], axis=1)
    f1_ref[...] = jnp.concatenate(blocks[3:], axis=1)
    pad10 = jnp.zeros((r, 10), jnp.float32)
    el_ref[...] = jnp.concatenate(els + [pad10], axis=1)
    er_ref[...] = jnp.concatenate(ers + [pad10], axis=1)
    xp_ref[...] = jnp.concatenate([x, ones, zer5], axis=1)


def _prep(x, gat_W, gat_al, gat_ar):
    rb = 1000
    grid = N_NODES // rb
    return pl.pallas_call(
        _prep_body,
        grid=(grid,),
        in_specs=[
            pl.BlockSpec((rb, D), lambda i: (i, 0)),
            pl.BlockSpec((D, H * D), lambda i: (0, 0)),
            pl.BlockSpec((H, D), lambda i: (0, 0)),
            pl.BlockSpec((H, D), lambda i: (0, 0)),
        ],
        out_specs=[
            pl.BlockSpec((rb, 3 * HP), lambda i: (i, 0)),
            pl.BlockSpec((rb, 3 * HP), lambda i: (i, 0)),
            pl.BlockSpec((rb, 16), lambda i: (i, 0)),
            pl.BlockSpec((rb, 16), lambda i: (i, 0)),
            pl.BlockSpec((rb, HP), lambda i: (i, 0)),
        ],
        out_shape=[
            jax.ShapeDtypeStruct((N_NODES, 3 * HP), jnp.float32),
            jax.ShapeDtypeStruct((N_NODES, 3 * HP), jnp.float32),
            jax.ShapeDtypeStruct((N_NODES, 16), jnp.float32),
            jax.ShapeDtypeStruct((N_NODES, 16), jnp.float32),
            jax.ShapeDtypeStruct((N_NODES, HP), jnp.float32),
        ],
    )(x, gat_W, gat_al, gat_ar)


def _sc_gat_body(src_hbm, dst_hbm, elp_hbm, erp_hbm, fcat_hbm, z144_hbm,
                 outcat_hbm,
                 srcva, dstva, adja, srcvb, dstvb, adjb,
                 elsa, ersa, elsb, ersb, featsa, featsb,
                 out_sh, g1a, g2a, g3a, g1b, g2b, g3b, ssa, ssb):
    cid = lax.axis_index("c")
    sid = lax.axis_index("s")

    rows = NP // NUM_SUBCORES
    r0 = sid * rows
    pltpu.sync_copy(z144_hbm.at[pl.ds(r0, rows)], out_sh.at[pl.ds(r0, rows)])
    plsc.subcore_barrier()

    h0 = 3 * cid
    zero16 = jnp.zeros((16,), jnp.int32)
    hidx = [(zero16 + (h0 + hh))[:, None] for hh in range(3)]
    dnums = lax.GatherDimensionNumbers(
        offset_dims=(), collapsed_slice_dims=(0,), start_index_map=(0,))
    row_off = cid * N_NODES
    out_off = cid * NP

    base_e = sid * EDGES_PER_SUBCORE

    def fire_gathers(cb, srcv, dstv, adj, els, ers, feats, s1, s2, s3):
        pltpu.sync_copy(src_hbm.at[pl.ds(cb, CHUNK)], srcv)
        pltpu.sync_copy(dst_hbm.at[pl.ds(cb, CHUNK)], dstv)
        for i in range(CHUNK // 16):
            adj[pl.ds(i * 16, 16)] = srcv[pl.ds(i * 16, 16)] + row_off
        cp1 = pltpu.async_copy(fcat_hbm.at[adj], feats, s1)
        cp2 = pltpu.async_copy(elp_hbm.at[srcv], els, s2)
        cp3 = pltpu.async_copy(erp_hbm.at[dstv], ers, s3)
        return cp1, cp2, cp3

    def compute(els, ers, feats):
        def edge_body(i4, c2):
            for u in range(4):
                i = 4 * i4 + u
                e = els[i] + ers[i]
                e = jnp.where(e >= 0.0, e, 0.2 * e)
                ex = jnp.exp(e)
                exb = [lax.gather(ex, hidx[hh], dnums, slice_sizes=(1,),
                                  mode=lax.GatherScatterMode.PROMISE_IN_BOUNDS)
                       for hh in range(3)]
                for j in range(9):
                    feats[i, pl.ds(16 * j, 16)] = (
                        exb[j // 3] * feats[i, pl.ds(16 * j, 16)])
            return c2

        lax.fori_loop(0, CHUNK // 4, edge_body, 0)

    def pair_body(k, carry):
        cb = base_e + 2 * k * CHUNK
        cpa = fire_gathers(cb, srcva, dstva, adja, elsa, ersa, featsa,
                           g1a, g2a, g3a)
        cpb = fire_gathers(cb + CHUNK, srcvb, dstvb, adjb, elsb, ersb, featsb,
                           g1b, g2b, g3b)
        for cp in cpa:
            cp.wait()
        compute(elsa, ersa, featsa)
        sa = pltpu.async_copy(featsa, out_sh.at[dstva], ssa, add=True)
        for cp in cpb:
            cp.wait()
        compute(elsb, ersb, featsb)
        sb = pltpu.async_copy(featsb, out_sh.at[dstvb], ssb, add=True)
        sa.wait()
        sb.wait()
        return carry

    lax.fori_loop(0, NCHUNKS // 2, pair_body, 0)
    plsc.subcore_barrier()

    pltpu.sync_copy(out_sh.at[pl.ds(r0, rows)],
                    outcat_hbm.at[pl.ds(out_off + r0, rows)])


def _sc_gin_body(src_hbm, dst_hbm, xp_hbm, z48_hbm,
                 agg_hbm,
                 srcva, dstva, xsa, srcvb, dstvb, xsb,
                 agg_sh, ga, gb, ssa, ssb):
    cid = lax.axis_index("c")
    sid = lax.axis_index("s")

    rows = NP // NUM_SUBCORES
    r0 = sid * rows
    pltpu.sync_copy(z48_hbm.at[pl.ds(r0, rows)], agg_sh.at[pl.ds(r0, rows)])
    plsc.subcore_barrier()

    half = N_EDGES // 2
    ept = half // NUM_SUBCORES
    base_e = cid * half + sid * ept
    out_off = cid * NP

    def pair_body(k, carry):
        cb = base_e + 2 * k * CHUNK
        pltpu.sync_copy(src_hbm.at[pl.ds(cb, CHUNK)], srcva)
        pltpu.sync_copy(dst_hbm.at[pl.ds(cb, CHUNK)], dstva)
        cpa = pltpu.async_copy(xp_hbm.at[srcva], xsa, ga)
        pltpu.sync_copy(src_hbm.at[pl.ds(cb + CHUNK, CHUNK)], srcvb)
        pltpu.sync_copy(dst_hbm.at[pl.ds(cb + CHUNK, CHUNK)], dstvb)
        cpb = pltpu.async_copy(xp_hbm.at[srcvb], xsb, gb)
        cpa.wait()
        sa = pltpu.async_copy(xsa, agg_sh.at[dstva], ssa, add=True)
        cpb.wait()
        sb = pltpu.async_copy(xsb, agg_sh.at[dstvb], ssb, add=True)
        sa.wait()
        sb.wait()
        return carry

    npairs = ept // (2 * CHUNK)
    lax.fori_loop(0, npairs, pair_body, 0)
    tb = base_e + 2 * npairs * CHUNK
    pltpu.sync_copy(src_hbm.at[pl.ds(tb, CHUNK)], srcva)
    pltpu.sync_copy(dst_hbm.at[pl.ds(tb, CHUNK)], dstva)
    pltpu.async_copy(xp_hbm.at[srcva], xsa, ga).wait()
    pltpu.sync_copy(xsa, agg_sh.at[dstva], add=True)
    plsc.subcore_barrier()

    pltpu.sync_copy(agg_sh.at[pl.ds(r0, rows)],
                    agg_hbm.at[pl.ds(out_off + r0, rows)])


def _sc_edge(src, dst, elp, erp, fcat, xp, z144, z48):
    mesh = plsc.VectorSubcoreMesh(core_axis_name="c", subcore_axis_name="s")
    gat = functools.partial(
        pl.kernel,
        mesh=mesh,
        compiler_params=pltpu.CompilerParams(use_tc_tiling_on_sc=False),
        out_type=jax.ShapeDtypeStruct((2 * NP, 3 * HP), jnp.float32),
        scratch_types=(
            [pltpu.VMEM((CHUNK,), jnp.int32)] * 6
            + [pltpu.VMEM((CHUNK, 16), jnp.float32)] * 4
            + [pltpu.VMEM((CHUNK, 3 * HP), jnp.float32)] * 2
            + [pltpu.VMEM_SHARED((NP, 3 * HP), jnp.float32)]
            + [pltpu.SemaphoreType.DMA] * 8
        ),
    )(_sc_gat_body)
    gin = functools.partial(
        pl.kernel,
        mesh=mesh,
        compiler_params=pltpu.CompilerParams(use_tc_tiling_on_sc=False),
        out_type=jax.ShapeDtypeStruct((2 * NP, HP), jnp.float32),
        scratch_types=(
            [pltpu.VMEM((CHUNK,), jnp.int32)] * 2
            + [pltpu.VMEM((CHUNK, HP), jnp.float32)]
            + [pltpu.VMEM((CHUNK,), jnp.int32)] * 2
            + [pltpu.VMEM((CHUNK, HP), jnp.float32)]
            + [pltpu.VMEM_SHARED((NP, HP), jnp.float32)]
            + [pltpu.SemaphoreType.DMA] * 4
        ),
    )(_sc_gin_body)
    outcat = gat(src, dst, elp, erp, fcat, z144)
    aggcat = gin(src, dst, xp, z48)
    return outcat, aggcat


def _nodes_body(x_ref, agg_ref, out0_ref, out1_ref, ginw_ref, ginb_ref,
                gatb_ref, z_ref):
    x = x_ref[...]
    agg = agg_ref[...]
    deg = agg[:, D:D + 1]
    mean = jnp.where(deg > 0.0, agg[:, :D] / jnp.maximum(deg, 1.0), 0.0)
    gin = jnp.dot(x + mean, ginw_ref[...],
                  preferred_element_type=jnp.float32) + ginb_ref[...][None, :]
    gatb = gatb_ref[...]
    for t in range(H):
        oref = out0_ref if t < 3 else out1_ref
        c0 = HP * (t % 3)
        blk = oref[:, c0:c0 + HP]
        denom = jnp.maximum(blk[:, D:D + 1], 1e-9)
        gat = blk[:, :D] / denom + gatb[t][None, :]
        z_ref[t] = jnp.concatenate([gin, gat], axis=1)


def _nodes(x, agg, out0, out1, gin_W, gin_b, gat_b):
    rb = 1000
    return pl.pallas_call(
        _nodes_body,
        grid=(N_NODES // rb,),
        in_specs=[
            pl.BlockSpec((rb, D), lambda i: (i, 0)),
            pl.BlockSpec((rb, HP), lambda i: (i, 0)),
            pl.BlockSpec((rb, 3 * HP), lambda i: (i, 0)),
            pl.BlockSpec((rb, 3 * HP), lambda i: (i, 0)),
            pl.BlockSpec((D, D), lambda i: (0, 0)),
            pl.BlockSpec((D,), lambda i: (0,)),
            pl.BlockSpec((H, D), lambda i: (0, 0)),
        ],
        out_specs=pl.BlockSpec((H, rb, 2 * D), lambda i: (0, i, 0)),
        out_shape=jax.ShapeDtypeStruct((H, N_NODES, 2 * D), jnp.float32),
    )(x, agg, out0, out1, gin_W, gin_b, gat_b)


def _pool_body(z_ref, gid_col_ref, gid_row_ref,
               wih_ref, whh_ref, bih_ref, bhh_ref,
               tw1_ref, tb1_ref, tw2_ref, tb2_ref, tw3_ref, tb3_ref,
               y_ref):
    z = z_ref[0]
    gid_col = gid_col_ref[...]
    gid_row = gid_row_ref[...]
    oh_b = lax.broadcasted_iota(jnp.int32, (N_NODES, NUM_GRAPHS), 1) == gid_col
    oh = oh_b.astype(jnp.float32)
    oht = (lax.broadcasted_iota(jnp.int32, (NUM_GRAPHS, N_NODES), 0)
           == gid_row).astype(jnp.float32)

    wih = wih_ref[...]
    whh = whh_ref[...]
    bih = bih_ref[...][None, :]
    bhh = bhh_ref[...][None, :]
    neg_inf = jnp.float32(-jnp.inf)

    h = jnp.zeros((NUM_GRAPHS, S2S), jnp.float32)
    c = jnp.zeros((NUM_GRAPHS, S2S), jnp.float32)
    q_star = jnp.zeros((NUM_GRAPHS, 2 * S2S), jnp.float32)
    for _ in range(2):
        gates = (lax.dot_general(q_star, wih, (((1,), (1,)), ((), ())),
                                 preferred_element_type=jnp.float32)
                 + bih
                 + lax.dot_general(h, whh, (((1,), (1,)), ((), ())),
                                   preferred_element_type=jnp.float32)
                 + bhh)
        ig = jax.nn.sigmoid(gates[:, :S2S])
        fg = jax.nn.sigmoid(gates[:, S2S:2 * S2S])
        gg = jnp.tanh(gates[:, 2 * S2S:3 * S2S])
        og = jax.nn.sigmoid(gates[:, 3 * S2S:])
        c = fg * c + ig * gg
        h = og * jnp.tanh(c)
        q = h
        e1 = lax.dot_general(z, q, (((1,), (1,)), ((), ())),
                             preferred_element_type=jnp.float32)
        e = jnp.sum(e1 * oh, axis=1, keepdims=True)
        m = jnp.max(jnp.where(oh_b, e1, neg_inf), axis=0,
                    keepdims=True)
        m = jnp.where(jnp.isfinite(m), m, 0.0)
        mgid = jnp.sum(oh * m, axis=1, keepdims=True)
        ex = jnp.exp(e - mgid)
        s = jnp.sum(oh * ex, axis=0, keepdims=True)
        sgid = jnp.sum(oh * s, axis=1, keepdims=True)
        alpha = ex / jnp.maximum(sgid, 1e-9)
        r = jnp.dot(oht, alpha * z,
                    preferred_element_type=jnp.float32)
        q_star = jnp.concatenate([q, r], axis=1)

    h1 = q_star @ tw1_ref[0] + tb1_ref[0]
    h1 = jnp.where(h1 > 0.0, h1, jnp.exp(jnp.minimum(h1, 0.0)) - 1.0)
    h2 = h1 @ tw2_ref[0] + tb2_ref[0]
    h2 = jnp.where(h2 > 0.0, h2, jnp.exp(jnp.minimum(h2, 0.0)) - 1.0)
    y_ref[0] = h2 @ tw3_ref[0] + tb3_ref[0]


def _pool(zstack, gid_col, gid_row, lstm_Wih, lstm_Whh, lstm_bih, lstm_bhh,
          tw1, tb1, tw2, tb2, tw3, tb3):
    full = lambda *s: pl.BlockSpec(s, lambda t: tuple(0 for _ in s))
    per_task = lambda *s: pl.BlockSpec((1,) + s, lambda t: (t,) + tuple(
        0 for _ in s))
    return pl.pallas_call(
        _pool_body,
        grid=(H,),
        in_specs=[
            per_task(N_NODES, 2 * D),
            full(N_NODES, 1),
            full(1, N_NODES),
            full(4 * S2S, 2 * S2S),
            full(4 * S2S, S2S),
            full(4 * S2S),
            full(4 * S2S),
            per_task(2 * S2S, 32),
            per_task(1, 32),
            per_task(32, 16),
            per_task(1, 16),
            per_task(16, 1),
            per_task(1, 1),
        ],
        out_specs=per_task(NUM_GRAPHS, 1),
        out_shape=jax.ShapeDtypeStruct((H, NUM_GRAPHS, 1), jnp.float32),
    )(zstack, gid_col, gid_row, lstm_Wih, lstm_Whh, lstm_bih, lstm_bhh,
      tw1, tb1, tw2, tb2, tw3, tb3)


def kernel(x, edge_index, node_graph_ids, gin_W, gin_b, gat_W, gat_al, gat_ar,
           gat_b, lstm_Wih, lstm_Whh, lstm_bih, lstm_bhh,
           tw1, tb1, tw2, tb2, tw3, tb3):
    src = edge_index[0]
    dst = edge_index[1]
    f0, f1, elp, erp, xp = _prep(x, gat_W, gat_al, gat_ar)
    fcat = jnp.concatenate([f0, f1], axis=0)
    z144 = jnp.zeros((NP, 3 * HP), jnp.float32)
    z48 = jnp.zeros((NP, HP), jnp.float32)
    outcat, aggcat = _sc_edge(src, dst, elp, erp, fcat, xp, z144, z48)
    out0 = outcat[:N_NODES]
    out1 = outcat[NP:NP + N_NODES]
    agg = aggcat[:N_NODES] + aggcat[NP:NP + N_NODES]
    gid_col = node_graph_ids.reshape(N_NODES, 1)
    gid_row = node_graph_ids.reshape(1, N_NODES)
    zstack = _nodes(x, agg, out0, out1, gin_W, gin_b, gat_b)
    y = _pool(zstack, gid_col, gid_row, lstm_Wih, lstm_Whh, lstm_bih,
              lstm_bhh, tw1, tb1.reshape(H, 1, 32), tw2, tb2.reshape(H, 1, 16),
              tw3, tb3.reshape(H, 1, 1))
    return y.reshape(H * NUM_GRAPHS, 1)
